# Initial kernel scaffold; baseline (speedup 1.0000x reference)
import functools

import jax
import jax.numpy as jnp
from jax import lax
from jax.experimental import pallas as pl
from jax.experimental.pallas import tpu as pltpu
from jax.experimental.pallas import tpu_sc as plsc

L = 16
G = 128


@functools.cache
def _make_sc_bilinear(N, HW, C, W, H):
    info = plsc.get_sparse_core_info()
    NW = info.num_cores * info.num_subcores
    npw = N // NW
    nchunks = npw // G
    mesh = plsc.VectorSubcoreMesh(core_axis_name="c", subcore_axis_name="s")

    @functools.partial(
        pl.kernel,
        mesh=mesh,
        out_type=jax.ShapeDtypeStruct((N, C), jnp.float32),
        scratch_types=[
            pltpu.VMEM((G,), jnp.float32),
            pltpu.VMEM((G,), jnp.float32),
            pltpu.VMEM((G,), jnp.int32),
            pltpu.VMEM((G,), jnp.int32),
            pltpu.VMEM((G,), jnp.int32),
            pltpu.VMEM((G,), jnp.int32),
            pltpu.VMEM((G,), jnp.float32),
            pltpu.VMEM((G,), jnp.float32),
            pltpu.VMEM((G,), jnp.float32),
            pltpu.VMEM((G,), jnp.float32),
            pltpu.VMEM((G, C), jnp.float32),
            pltpu.VMEM((G, C), jnp.float32),
            pltpu.VMEM((G, C), jnp.float32),
            pltpu.VMEM((G, C), jnp.float32),
            pltpu.VMEM((G, C), jnp.float32),
            pltpu.SemaphoreType.DMA,
        ],
    )
    def sc_kernel(xs_hbm, ys_hbm, table_hbm, out_hbm,
                  xv, yv, i00, i01, i10, i11, w00, w01, w10, w11,
                  r00, r01, r10, r11, ov, sem):
        wid = lax.axis_index("s") * info.num_cores + lax.axis_index("c")
        base0 = wid * npw

        def chunk(ci, carry):
            base = base0 + ci * G
            pltpu.sync_copy(xs_hbm.at[pl.ds(base, G)], xv)
            pltpu.sync_copy(ys_hbm.at[pl.ds(base, G)], yv)

            def grp(j, c2):
                s = pl.ds(j * L, L)
                x = xv[s]
                y = yv[s]
                ix = jnp.clip((x + 1.0) * (0.5 * (W - 1)), 0.0, float(W - 1))
                iy = jnp.clip((y + 1.0) * (0.5 * (H - 1)), 0.0, float(H - 1))
                x0 = ix.astype(jnp.int32)
                y0 = iy.astype(jnp.int32)
                fx = ix - x0.astype(jnp.float32)
                fy = iy - y0.astype(jnp.float32)
                x1 = jnp.minimum(x0 + 1, W - 1)
                y1 = jnp.minimum(y0 + 1, H - 1)
                b0 = y0 * W
                b1 = y1 * W
                i00[s] = b0 + x0
                i01[s] = b0 + x1
                i10[s] = b1 + x0
                i11[s] = b1 + x1
                gx = 1.0 - fx
                gy = 1.0 - fy
                w00[s] = gx * gy
                w01[s] = fx * gy
                w10[s] = gx * fy
                w11[s] = fx * fy
                return c2

            lax.fori_loop(0, G // L, grp, 0)

            cp0 = pltpu.async_copy(table_hbm.at[i00], r00, sem)
            cp1 = pltpu.async_copy(table_hbm.at[i01], r01, sem)
            cp2 = pltpu.async_copy(table_hbm.at[i10], r10, sem)
            cp3 = pltpu.async_copy(table_hbm.at[i11], r11, sem)
            cp0.wait()
            cp1.wait()
            cp2.wait()
            cp3.wait()

            iota = lax.iota(jnp.int32, L)

            def grp2(j, c2):
                s = pl.ds(j * L, L)
                rows = j * L + iota
                a00 = w00[s]
                a01 = w01[s]
                a10 = w10[s]
                a11 = w11[s]
                for c in range(C):
                    col = jnp.full((L,), c, jnp.int32)
                    v00 = plsc.load_gather(r00, [rows, col])
                    v01 = plsc.load_gather(r01, [rows, col])
                    v10 = plsc.load_gather(r10, [rows, col])
                    v11 = plsc.load_gather(r11, [rows, col])
                    acc = a00 * v00 + a01 * v01 + a10 * v10 + a11 * v11
                    plsc.store_scatter(ov, [rows, col], acc)
                return c2

            lax.fori_loop(0, G // L, grp2, 0)

            pltpu.sync_copy(ov, out_hbm.at[pl.ds(base, G), :])
            return carry

        lax.fori_loop(0, nchunks, chunk, 0)

    return sc_kernel


def kernel(inp, plane):
    C, H, W = plane.shape
    N = inp.shape[0]
    table = plane.transpose(1, 2, 0).reshape(H * W, C)
    xs = inp[:, 0]
    ys = inp[:, 1]
    return _make_sc_bilinear(N, H * W, C, W, H)(xs, ys, table)


# speedup vs baseline: 2.9278x; 2.9278x over previous
import functools

import jax
import jax.numpy as jnp
from jax import lax
from jax.experimental import pallas as pl
from jax.experimental.pallas import tpu as pltpu
from jax.experimental.pallas import tpu_sc as plsc

L = 16
G = 128


@functools.cache
def _make_sc_bilinear(N, HW, C, W, H):
    info = plsc.get_sparse_core_info()
    NW = info.num_cores * info.num_subcores
    npw = N // NW
    nchunks = npw // G
    mesh = plsc.VectorSubcoreMesh(core_axis_name="c", subcore_axis_name="s")

    @functools.partial(
        pl.kernel,
        mesh=mesh,
        compiler_params=pltpu.CompilerParams(use_tc_tiling_on_sc=False),
        out_type=jax.ShapeDtypeStruct((N, C), jnp.float32),
        scratch_types=[
            pltpu.VMEM((G,), jnp.float32),
            pltpu.VMEM((G,), jnp.float32),
            pltpu.VMEM((G,), jnp.int32),
            pltpu.VMEM((G,), jnp.int32),
            pltpu.VMEM((G,), jnp.int32),
            pltpu.VMEM((G,), jnp.int32),
            pltpu.VMEM((G,), jnp.float32),
            pltpu.VMEM((G,), jnp.float32),
            pltpu.VMEM((G,), jnp.float32),
            pltpu.VMEM((G,), jnp.float32),
            pltpu.VMEM((G, C), jnp.float32),
            pltpu.VMEM((G, C), jnp.float32),
            pltpu.VMEM((G, C), jnp.float32),
            pltpu.VMEM((G, C), jnp.float32),
            pltpu.VMEM((G, C), jnp.float32),
            pltpu.SemaphoreType.DMA,
        ],
    )
    def sc_kernel(xs_hbm, ys_hbm, table_hbm, out_hbm,
                  xv, yv, i00, i01, i10, i11, w00, w01, w10, w11,
                  r00, r01, r10, r11, ov, sem):
        wid = lax.axis_index("s") * info.num_cores + lax.axis_index("c")
        base0 = wid * npw

        def chunk(ci, carry):
            base = base0 + ci * G
            pltpu.sync_copy(xs_hbm.at[pl.ds(base, G)], xv)
            pltpu.sync_copy(ys_hbm.at[pl.ds(base, G)], yv)

            def grp(j, c2):
                s = pl.ds(j * L, L)
                x = xv[s]
                y = yv[s]
                ix = jnp.clip((x + 1.0) * (0.5 * (W - 1)), 0.0, float(W - 1))
                iy = jnp.clip((y + 1.0) * (0.5 * (H - 1)), 0.0, float(H - 1))
                x0 = ix.astype(jnp.int32)
                y0 = iy.astype(jnp.int32)
                fx = ix - x0.astype(jnp.float32)
                fy = iy - y0.astype(jnp.float32)
                x1 = jnp.minimum(x0 + 1, W - 1)
                y1 = jnp.minimum(y0 + 1, H - 1)
                b0 = y0 * W
                b1 = y1 * W
                i00[s] = b0 + x0
                i01[s] = b0 + x1
                i10[s] = b1 + x0
                i11[s] = b1 + x1
                gx = 1.0 - fx
                gy = 1.0 - fy
                w00[s] = gx * gy
                w01[s] = fx * gy
                w10[s] = gx * fy
                w11[s] = fx * fy
                return c2

            lax.fori_loop(0, G // L, grp, 0)

            cp0 = pltpu.async_copy(table_hbm.at[i00], r00, sem)
            cp1 = pltpu.async_copy(table_hbm.at[i01], r01, sem)
            cp2 = pltpu.async_copy(table_hbm.at[i10], r10, sem)
            cp3 = pltpu.async_copy(table_hbm.at[i11], r11, sem)
            cp0.wait()
            cp1.wait()
            cp2.wait()
            cp3.wait()

            def grp2(j, c2):
                sw = pl.ds(j * L, L)
                a00g = w00[sw]
                a01g = w01[sw]
                a10g = w10[sw]
                a11g = w11[sw]
                for k in range(L):
                    p = j * L + k
                    a00 = a00g[k]
                    a01 = a01g[k]
                    a10 = a10g[k]
                    a11 = a11g[k]
                    for c in range(0, C, L):
                        s = pl.ds(c, L)
                        acc = (a00 * r00[p, s] + a01 * r01[p, s]
                               + a10 * r10[p, s] + a11 * r11[p, s])
                        ov[p, s] = acc
                return c2

            lax.fori_loop(0, G // L, grp2, 0)

            pltpu.sync_copy(ov, out_hbm.at[pl.ds(base, G), :])
            return carry

        lax.fori_loop(0, nchunks, chunk, 0)

    return sc_kernel


def kernel(inp, plane):
    C, H, W = plane.shape
    N = inp.shape[0]
    table = plane.transpose(1, 2, 0).reshape(H * W, C)
    xs = inp[:, 0]
    ys = inp[:, 1]
    return _make_sc_bilinear(N, H * W, C, W, H)(xs, ys, table)
